# Initial kernel scaffold; baseline (speedup 1.0000x reference)
import jax
import jax.numpy as jnp
from jax.experimental import pallas as pl


def kernel(node_feat, adj_matrix, W1, b1, W2, b2, W3, b3, Wh1, bh1, Wh2, bh2):
    raise NotImplementedError("write your pallas kernel here")


# speedup vs baseline: 1.0645x; 1.0645x over previous
import functools

import jax
import jax.numpy as jnp
from jax.experimental import pallas as pl
from jax.experimental.pallas import tpu as pltpu

N = 8192
_CAST_BM = 256
_LAYER_BM = 256


def _cast_deg_kernel(adj_ref, adj16_ref, d_ref):
    blk = adj_ref[...]
    adj16_ref[...] = blk.astype(jnp.bfloat16)
    deg = jnp.sum(blk, axis=1, keepdims=True) + 1.0
    d_ref[...] = jax.lax.rsqrt(jnp.maximum(deg, 1e-12))


def _compute_z(h_ref, w_ref, d_ref, z_ref, zb_ref):
    z = d_ref[...] * jnp.dot(h_ref[...], w_ref[...],
                             preferred_element_type=jnp.float32)
    z_ref[...] = z
    zb_ref[...] = z.astype(jnp.bfloat16)


def _layer_step(adj16_ref, b_ref, d_ref, z_ref, zb_ref, bm):
    i = pl.program_id(0)
    row0 = i * bm
    t = jnp.dot(adj16_ref[...], zb_ref[...],
                preferred_element_type=jnp.float32)
    t = t + z_ref[pl.ds(row0, bm), :]
    di = d_ref[pl.ds(row0, bm), :]
    return jnp.maximum(di * t + b_ref[...], 0.0)


def _layer_kernel(adj16_ref, h_ref, w_ref, b_ref, d_ref, out_ref,
                  z_ref, zb_ref, *, bm):
    @pl.when(pl.program_id(0) == 0)
    def _():
        _compute_z(h_ref, w_ref, d_ref, z_ref, zb_ref)

    out_ref[...] = _layer_step(adj16_ref, b_ref, d_ref, z_ref, zb_ref, bm)


def _layer3_head_kernel(adj16_ref, h_ref, w_ref, b_ref, d_ref,
                        wh1_ref, bh1_ref, wh2_ref, bh2_ref,
                        logits_ref, probs_ref,
                        z_ref, zb_ref, emb_ref, *, bm, ni):
    @pl.when(pl.program_id(0) == 0)
    def _():
        _compute_z(h_ref, w_ref, d_ref, z_ref, zb_ref)
        emb_ref[...] = jnp.zeros_like(emb_ref)

    y = _layer_step(adj16_ref, b_ref, d_ref, z_ref, zb_ref, bm)
    emb_ref[...] += jnp.sum(y, axis=0, keepdims=True)

    @pl.when(pl.program_id(0) == ni - 1)
    def _():
        emb = emb_ref[...] * (1.0 / N)
        h1 = jnp.dot(emb, wh1_ref[...],
                     preferred_element_type=jnp.float32) + bh1_ref[...]
        u = jnp.exp(h1)
        lg = jnp.log(jnp.where(u == 1.0, 2.0, u))
        em1 = jnp.where(u == 1.0, h1, (u - 1.0) * h1 / lg)
        h1 = jnp.where(h1 > 0, h1, em1)
        logits = jnp.dot(h1, wh2_ref[...],
                         preferred_element_type=jnp.float32) + bh2_ref[...]
        logits_ref[...] = logits
        m = jnp.max(logits, axis=1, keepdims=True)
        e = jnp.exp(logits - m)
        probs_ref[...] = e / jnp.sum(e, axis=1, keepdims=True)


def _full(shape):
    return pl.BlockSpec(shape, lambda i: tuple(0 for _ in shape))


def _layer(adj16, h, w, b, d, dout):
    bm = _LAYER_BM
    ni = N // bm
    return pl.pallas_call(
        functools.partial(_layer_kernel, bm=bm),
        grid=(ni,),
        in_specs=[
            pl.BlockSpec((bm, N), lambda i: (i, 0)),
            _full(h.shape),
            _full(w.shape),
            _full(b.shape),
            _full((N, 1)),
        ],
        out_specs=pl.BlockSpec((bm, dout), lambda i: (i, 0)),
        out_shape=jax.ShapeDtypeStruct((N, dout), jnp.float32),
        scratch_shapes=[
            pltpu.VMEM((N, dout), jnp.float32),
            pltpu.VMEM((N, dout), jnp.bfloat16),
        ],
    )(adj16, h, w, b, d)


def _layer3_head(adj16, h, w, b, d, wh1, bh1, wh2, bh2, dout):
    bm = _LAYER_BM
    ni = N // bm
    return pl.pallas_call(
        functools.partial(_layer3_head_kernel, bm=bm, ni=ni),
        grid=(ni,),
        in_specs=[
            pl.BlockSpec((bm, N), lambda i: (i, 0)),
            _full(h.shape),
            _full(w.shape),
            _full(b.shape),
            _full((N, 1)),
            _full(wh1.shape),
            _full(bh1.shape),
            _full(wh2.shape),
            _full(bh2.shape),
        ],
        out_specs=[_full((1, 2)), _full((1, 2))],
        out_shape=[
            jax.ShapeDtypeStruct((1, 2), jnp.float32),
            jax.ShapeDtypeStruct((1, 2), jnp.float32),
        ],
        scratch_shapes=[
            pltpu.VMEM((N, dout), jnp.float32),
            pltpu.VMEM((N, dout), jnp.bfloat16),
            pltpu.VMEM((1, dout), jnp.float32),
        ],
    )(adj16, h, w, b, d, wh1, bh1, wh2, bh2)


def kernel(node_feat, adj_matrix, W1, b1, W2, b2, W3, b3, Wh1, bh1, Wh2, bh2):
    adj16, d = pl.pallas_call(
        _cast_deg_kernel,
        grid=(N // _CAST_BM,),
        in_specs=[pl.BlockSpec((_CAST_BM, N), lambda i: (i, 0))],
        out_specs=[
            pl.BlockSpec((_CAST_BM, N), lambda i: (i, 0)),
            pl.BlockSpec((_CAST_BM, 1), lambda i: (i, 0)),
        ],
        out_shape=[
            jax.ShapeDtypeStruct((N, N), jnp.bfloat16),
            jax.ShapeDtypeStruct((N, 1), jnp.float32),
        ],
    )(adj_matrix)

    h = _layer(adj16, node_feat, W1, b1.reshape(1, -1), d, W1.shape[1])
    h = _layer(adj16, h, W2, b2.reshape(1, -1), d, W2.shape[1])
    logits, probs = _layer3_head(
        adj16, h, W3, b3.reshape(1, -1), d,
        Wh1, bh1.reshape(1, -1), Wh2, bh2.reshape(1, -1), W3.shape[1])
    return (logits.reshape(2), probs.reshape(2))


# speedup vs baseline: 1.1809x; 1.1093x over previous
import functools

import jax
import jax.numpy as jnp
from jax.experimental import pallas as pl
from jax.experimental.pallas import tpu as pltpu

N = 8192
_DIN = 128
_DOUT = 64
_CAST_BM = 256
_BM = 512


def _cast_deg_kernel(adj_ref, adj16_ref, d_ref):
    blk = adj_ref[...]
    adj16_ref[...] = blk.astype(jnp.bfloat16)
    deg = jnp.sum(blk, axis=1, keepdims=True) + 1.0
    d_ref[...] = jax.lax.rsqrt(jnp.maximum(deg, 1e-12))


def _gcn_kernel(adj16_ref, nf_ref, w_ref, b_ref, d_ref,
                wh1_ref, bh1_ref, wh2_ref, bh2_ref,
                logits_ref, probs_ref,
                h_ref, z_ref, zb_ref, emb_ref, *, bm, ni):
    l = pl.program_id(0)
    i = pl.program_id(1)

    @pl.when((l == 0) & (i == 0))
    def _():
        h_ref[...] = nf_ref[...]
        emb_ref[...] = jnp.zeros_like(emb_ref)

    @pl.when(i == 0)
    def _():
        z = d_ref[...] * jnp.dot(h_ref[...], w_ref[0],
                                 preferred_element_type=jnp.float32)
        z_ref[...] = z
        zb_ref[...] = z.astype(jnp.bfloat16)

    row0 = i * bm
    t = jnp.dot(adj16_ref[...], zb_ref[...],
                preferred_element_type=jnp.float32)
    t = t + z_ref[pl.ds(row0, bm), :]
    di = d_ref[pl.ds(row0, bm), :]
    y = jnp.maximum(di * t + b_ref[0], 0.0)
    h_ref[pl.ds(row0, bm), :_DOUT] = y

    @pl.when(l == 2)
    def _():
        emb_ref[...] += jnp.sum(y, axis=0, keepdims=True)

    @pl.when((l == 2) & (i == ni - 1))
    def _():
        emb = emb_ref[...] * (1.0 / N)
        h1 = jnp.dot(emb, wh1_ref[...],
                     preferred_element_type=jnp.float32) + bh1_ref[...]
        u = jnp.exp(h1)
        lg = jnp.log(jnp.where(u == 1.0, 2.0, u))
        em1 = jnp.where(u == 1.0, h1, (u - 1.0) * h1 / lg)
        h1 = jnp.where(h1 > 0, h1, em1)
        logits = jnp.dot(h1, wh2_ref[...],
                         preferred_element_type=jnp.float32) + bh2_ref[...]
        logits_ref[...] = logits
        m = jnp.max(logits, axis=1, keepdims=True)
        e = jnp.exp(logits - m)
        probs_ref[...] = e / jnp.sum(e, axis=1, keepdims=True)


def _full(shape):
    return pl.BlockSpec(shape, lambda l, i: tuple(0 for _ in shape))


def _pad(w, rows, cols):
    return jnp.zeros((rows, cols), w.dtype).at[:w.shape[0], :w.shape[1]].set(w)


def kernel(node_feat, adj_matrix, W1, b1, W2, b2, W3, b3, Wh1, bh1, Wh2, bh2):
    adj16, d = pl.pallas_call(
        _cast_deg_kernel,
        grid=(N // _CAST_BM,),
        in_specs=[pl.BlockSpec((_CAST_BM, N), lambda i: (i, 0))],
        out_specs=[
            pl.BlockSpec((_CAST_BM, N), lambda i: (i, 0)),
            pl.BlockSpec((_CAST_BM, 1), lambda i: (i, 0)),
        ],
        out_shape=[
            jax.ShapeDtypeStruct((N, N), jnp.bfloat16),
            jax.ShapeDtypeStruct((N, 1), jnp.float32),
        ],
    )(adj_matrix)

    w_stack = jnp.stack([_pad(W1, _DIN, _DOUT), _pad(W2, _DIN, _DOUT),
                         _pad(W3, _DIN, _DOUT)])
    b_stack = jnp.stack([_pad(b1.reshape(1, -1), 1, _DOUT),
                         _pad(b2.reshape(1, -1), 1, _DOUT),
                         _pad(b3.reshape(1, -1), 1, _DOUT)])

    ni = N // _BM
    wh1 = _pad(Wh1, _DOUT, Wh1.shape[1])
    logits, probs = pl.pallas_call(
        functools.partial(_gcn_kernel, bm=_BM, ni=ni),
        grid=(3, ni),
        in_specs=[
            pl.BlockSpec((_BM, N), lambda l, i: (i, 0)),
            _full((N, _DIN)),
            pl.BlockSpec((1, _DIN, _DOUT), lambda l, i: (l, 0, 0)),
            pl.BlockSpec((1, 1, _DOUT), lambda l, i: (l, 0, 0)),
            _full((N, 1)),
            _full(wh1.shape),
            _full((1, Wh1.shape[1])),
            _full(Wh2.shape),
            _full((1, 2)),
        ],
        out_specs=[_full((1, 2)), _full((1, 2))],
        out_shape=[
            jax.ShapeDtypeStruct((1, 2), jnp.float32),
            jax.ShapeDtypeStruct((1, 2), jnp.float32),
        ],
        scratch_shapes=[
            pltpu.VMEM((N, _DIN), jnp.float32),
            pltpu.VMEM((N, _DOUT), jnp.float32),
            pltpu.VMEM((N, _DOUT), jnp.bfloat16),
            pltpu.VMEM((1, _DOUT), jnp.float32),
        ],
    )(adj16, node_feat, w_stack, b_stack, d,
      wh1, bh1.reshape(1, -1), Wh2, bh2.reshape(1, -1))

    return (logits.reshape(2), probs.reshape(2))


# speedup vs baseline: 1.3921x; 1.1789x over previous
import functools

import jax
import jax.numpy as jnp
from jax.experimental import pallas as pl
from jax.experimental.pallas import tpu as pltpu

N = 8192
_DIN = 128
_DOUT = 64
_CAST_BM = 256
_BM = 512


def _cast_deg_kernel(adj_ref, adj8_ref, d_ref):
    blk = adj_ref[...]
    adj8_ref[...] = blk.astype(jnp.float8_e4m3fn)
    deg = jnp.sum(blk, axis=1, keepdims=True) + 1.0
    d_ref[...] = jax.lax.rsqrt(jnp.maximum(deg, 1e-12))


def _gcn_kernel(adj16_ref, nf_ref, w_ref, b_ref, d_ref,
                wh1_ref, bh1_ref, wh2_ref, bh2_ref,
                logits_ref, probs_ref,
                h_ref, z_ref, zb_ref, emb_ref, *, bm, ni):
    l = pl.program_id(0)
    i = pl.program_id(1)

    @pl.when((l == 0) & (i == 0))
    def _():
        h_ref[...] = nf_ref[...]
        emb_ref[...] = jnp.zeros_like(emb_ref)

    @pl.when(i == 0)
    def _():
        z = d_ref[...] * jnp.dot(h_ref[...], w_ref[0],
                                 preferred_element_type=jnp.float32)
        z_ref[...] = z
        zb_ref[...] = z.astype(jnp.bfloat16)

    row0 = i * bm
    t = jnp.dot(adj16_ref[...].astype(jnp.bfloat16), zb_ref[...],
                preferred_element_type=jnp.float32)
    t = t + z_ref[pl.ds(row0, bm), :]
    di = d_ref[pl.ds(row0, bm), :]
    y = jnp.maximum(di * t + b_ref[0], 0.0)
    h_ref[pl.ds(row0, bm), :_DOUT] = y

    @pl.when(l == 2)
    def _():
        emb_ref[...] += jnp.sum(y, axis=0, keepdims=True)

    @pl.when((l == 2) & (i == ni - 1))
    def _():
        emb = emb_ref[...] * (1.0 / N)
        h1 = jnp.dot(emb, wh1_ref[...],
                     preferred_element_type=jnp.float32) + bh1_ref[...]
        u = jnp.exp(h1)
        lg = jnp.log(jnp.where(u == 1.0, 2.0, u))
        em1 = jnp.where(u == 1.0, h1, (u - 1.0) * h1 / lg)
        h1 = jnp.where(h1 > 0, h1, em1)
        logits = jnp.dot(h1, wh2_ref[...],
                         preferred_element_type=jnp.float32) + bh2_ref[...]
        logits_ref[...] = logits
        m = jnp.max(logits, axis=1, keepdims=True)
        e = jnp.exp(logits - m)
        probs_ref[...] = e / jnp.sum(e, axis=1, keepdims=True)


def _full(shape):
    return pl.BlockSpec(shape, lambda l, i: tuple(0 for _ in shape))


def _pad(w, rows, cols):
    return jnp.zeros((rows, cols), w.dtype).at[:w.shape[0], :w.shape[1]].set(w)


def kernel(node_feat, adj_matrix, W1, b1, W2, b2, W3, b3, Wh1, bh1, Wh2, bh2):
    adj16, d = pl.pallas_call(
        _cast_deg_kernel,
        grid=(N // _CAST_BM,),
        in_specs=[pl.BlockSpec((_CAST_BM, N), lambda i: (i, 0))],
        out_specs=[
            pl.BlockSpec((_CAST_BM, N), lambda i: (i, 0)),
            pl.BlockSpec((_CAST_BM, 1), lambda i: (i, 0)),
        ],
        out_shape=[
            jax.ShapeDtypeStruct((N, N), jnp.float8_e4m3fn),
            jax.ShapeDtypeStruct((N, 1), jnp.float32),
        ],
    )(adj_matrix)

    w_stack = jnp.stack([_pad(W1, _DIN, _DOUT), _pad(W2, _DIN, _DOUT),
                         _pad(W3, _DIN, _DOUT)])
    b_stack = jnp.stack([_pad(b1.reshape(1, -1), 1, _DOUT),
                         _pad(b2.reshape(1, -1), 1, _DOUT),
                         _pad(b3.reshape(1, -1), 1, _DOUT)])

    ni = N // _BM
    wh1 = _pad(Wh1, _DOUT, Wh1.shape[1])
    logits, probs = pl.pallas_call(
        functools.partial(_gcn_kernel, bm=_BM, ni=ni),
        grid=(3, ni),
        in_specs=[
            pl.BlockSpec((_BM, N), lambda l, i: (i, 0)),
            _full((N, _DIN)),
            pl.BlockSpec((1, _DIN, _DOUT), lambda l, i: (l, 0, 0)),
            pl.BlockSpec((1, 1, _DOUT), lambda l, i: (l, 0, 0)),
            _full((N, 1)),
            _full(wh1.shape),
            _full((1, Wh1.shape[1])),
            _full(Wh2.shape),
            _full((1, 2)),
        ],
        out_specs=[_full((1, 2)), _full((1, 2))],
        out_shape=[
            jax.ShapeDtypeStruct((1, 2), jnp.float32),
            jax.ShapeDtypeStruct((1, 2), jnp.float32),
        ],
        scratch_shapes=[
            pltpu.VMEM((N, _DIN), jnp.float32),
            pltpu.VMEM((N, _DOUT), jnp.float32),
            pltpu.VMEM((N, _DOUT), jnp.bfloat16),
            pltpu.VMEM((1, _DOUT), jnp.float32),
        ],
    )(adj16, node_feat, w_stack, b_stack, d,
      wh1, bh1.reshape(1, -1), Wh2, bh2.reshape(1, -1))

    return (logits.reshape(2), probs.reshape(2))


# speedup vs baseline: 1.4149x; 1.0164x over previous
import functools

import jax
import jax.numpy as jnp
from jax.experimental import pallas as pl
from jax.experimental.pallas import tpu as pltpu

N = 8192
_DIN = 128
_DOUT = 64
_CAST_BM = 256
_BM = 512


def _cast_deg_kernel(adj_ref, adj8_ref, d_ref):
    blk = adj_ref[...]
    adj8_ref[...] = blk.astype(jnp.float8_e4m3fn)
    deg = jnp.sum(blk, axis=1, keepdims=True) + 1.0
    d_ref[...] = jax.lax.rsqrt(jnp.maximum(deg, 1e-12))


def _gcn_kernel(adj16_ref, nf_ref, w_ref, b_ref, d_ref,
                wh1_ref, bh1_ref, wh2_ref, bh2_ref,
                logits_ref, probs_ref,
                h_ref, z_ref, zb_ref, emb_ref, *, bm, ni):
    l = pl.program_id(0)
    i = pl.program_id(1)

    @pl.when((l == 0) & (i == 0))
    def _():
        h_ref[...] = nf_ref[...]
        emb_ref[...] = jnp.zeros_like(emb_ref)

    @pl.when(i == 0)
    def _():
        z = d_ref[...] * jnp.dot(h_ref[...], w_ref[0],
                                 preferred_element_type=jnp.float32)
        z_ref[...] = z
        zb_ref[...] = z.astype(jnp.bfloat16)

    row0 = i * bm
    t = jax.lax.dot_general(
        adj16_ref[...], zb_ref[...], (((1,), (0,)), ((), ())),
        preferred_element_type=jnp.float32)
    t = t + z_ref[pl.ds(row0, bm), :]
    di = d_ref[pl.ds(row0, bm), :]
    y = jnp.maximum(di * t + b_ref[0], 0.0)
    h_ref[pl.ds(row0, bm), :_DOUT] = y

    @pl.when(l == 2)
    def _():
        emb_ref[...] += jnp.sum(y, axis=0, keepdims=True)

    @pl.when((l == 2) & (i == ni - 1))
    def _():
        emb = emb_ref[...] * (1.0 / N)
        h1 = jnp.dot(emb, wh1_ref[...],
                     preferred_element_type=jnp.float32) + bh1_ref[...]
        u = jnp.exp(h1)
        lg = jnp.log(jnp.where(u == 1.0, 2.0, u))
        em1 = jnp.where(u == 1.0, h1, (u - 1.0) * h1 / lg)
        h1 = jnp.where(h1 > 0, h1, em1)
        logits = jnp.dot(h1, wh2_ref[...],
                         preferred_element_type=jnp.float32) + bh2_ref[...]
        logits_ref[...] = logits
        m = jnp.max(logits, axis=1, keepdims=True)
        e = jnp.exp(logits - m)
        probs_ref[...] = e / jnp.sum(e, axis=1, keepdims=True)


def _full(shape):
    return pl.BlockSpec(shape, lambda l, i: tuple(0 for _ in shape))


def _pad(w, rows, cols):
    return jnp.zeros((rows, cols), w.dtype).at[:w.shape[0], :w.shape[1]].set(w)


def kernel(node_feat, adj_matrix, W1, b1, W2, b2, W3, b3, Wh1, bh1, Wh2, bh2):
    adj16, d = pl.pallas_call(
        _cast_deg_kernel,
        grid=(N // _CAST_BM,),
        in_specs=[pl.BlockSpec((_CAST_BM, N), lambda i: (i, 0))],
        out_specs=[
            pl.BlockSpec((_CAST_BM, N), lambda i: (i, 0)),
            pl.BlockSpec((_CAST_BM, 1), lambda i: (i, 0)),
        ],
        out_shape=[
            jax.ShapeDtypeStruct((N, N), jnp.float8_e4m3fn),
            jax.ShapeDtypeStruct((N, 1), jnp.float32),
        ],
    )(adj_matrix)

    w_stack = jnp.stack([_pad(W1, _DIN, _DOUT), _pad(W2, _DIN, _DOUT),
                         _pad(W3, _DIN, _DOUT)])
    b_stack = jnp.stack([_pad(b1.reshape(1, -1), 1, _DOUT),
                         _pad(b2.reshape(1, -1), 1, _DOUT),
                         _pad(b3.reshape(1, -1), 1, _DOUT)])

    ni = N // _BM
    wh1 = _pad(Wh1, _DOUT, Wh1.shape[1])
    logits, probs = pl.pallas_call(
        functools.partial(_gcn_kernel, bm=_BM, ni=ni),
        grid=(3, ni),
        in_specs=[
            pl.BlockSpec((_BM, N), lambda l, i: (i, 0)),
            _full((N, _DIN)),
            pl.BlockSpec((1, _DIN, _DOUT), lambda l, i: (l, 0, 0)),
            pl.BlockSpec((1, 1, _DOUT), lambda l, i: (l, 0, 0)),
            _full((N, 1)),
            _full(wh1.shape),
            _full((1, Wh1.shape[1])),
            _full(Wh2.shape),
            _full((1, 2)),
        ],
        out_specs=[_full((1, 2)), _full((1, 2))],
        out_shape=[
            jax.ShapeDtypeStruct((1, 2), jnp.float32),
            jax.ShapeDtypeStruct((1, 2), jnp.float32),
        ],
        scratch_shapes=[
            pltpu.VMEM((N, _DIN), jnp.float32),
            pltpu.VMEM((N, _DOUT), jnp.float32),
            pltpu.VMEM((N, _DOUT), jnp.bfloat16),
            pltpu.VMEM((1, _DOUT), jnp.float32),
        ],
    )(adj16, node_feat, w_stack, b_stack, d,
      wh1, bh1.reshape(1, -1), Wh2, bh2.reshape(1, -1))

    return (logits.reshape(2), probs.reshape(2))


# speedup vs baseline: 1.4176x; 1.0019x over previous
import functools

import jax
import jax.numpy as jnp
from jax.experimental import pallas as pl
from jax.experimental.pallas import tpu as pltpu

N = 8192
_DIN = 128
_DOUT = 64
_CAST_BM = 256
_BM = 1024


def _cast_deg_kernel(adj_ref, adj8_ref, d_ref):
    blk = adj_ref[...]
    adj8_ref[...] = blk.astype(jnp.float8_e4m3fn)
    deg = jnp.sum(blk, axis=1, keepdims=True) + 1.0
    d_ref[...] = jax.lax.rsqrt(jnp.maximum(deg, 1e-12))


def _gcn_kernel(adj16_ref, nf_ref, w_ref, b_ref, d_ref,
                wh1_ref, bh1_ref, wh2_ref, bh2_ref,
                logits_ref, probs_ref,
                h_ref, z_ref, zb_ref, emb_ref, *, bm, ni):
    l = pl.program_id(0)
    i = pl.program_id(1)

    @pl.when((l == 0) & (i == 0))
    def _():
        h_ref[...] = nf_ref[...]
        emb_ref[...] = jnp.zeros_like(emb_ref)

    @pl.when(i == 0)
    def _():
        z = d_ref[...] * jnp.dot(h_ref[...], w_ref[0],
                                 preferred_element_type=jnp.float32)
        z_ref[...] = z
        zb_ref[...] = z.astype(jnp.bfloat16)

    row0 = i * bm
    t = jax.lax.dot_general(
        adj16_ref[...], zb_ref[...], (((1,), (0,)), ((), ())),
        preferred_element_type=jnp.float32)
    t = t + z_ref[pl.ds(row0, bm), :]
    di = d_ref[pl.ds(row0, bm), :]
    y = jnp.maximum(di * t + b_ref[0], 0.0)
    h_ref[pl.ds(row0, bm), :_DOUT] = y

    @pl.when(l == 2)
    def _():
        emb_ref[...] += jnp.sum(y, axis=0, keepdims=True)

    @pl.when((l == 2) & (i == ni - 1))
    def _():
        emb = emb_ref[...] * (1.0 / N)
        h1 = jnp.dot(emb, wh1_ref[...],
                     preferred_element_type=jnp.float32) + bh1_ref[...]
        u = jnp.exp(h1)
        lg = jnp.log(jnp.where(u == 1.0, 2.0, u))
        em1 = jnp.where(u == 1.0, h1, (u - 1.0) * h1 / lg)
        h1 = jnp.where(h1 > 0, h1, em1)
        logits = jnp.dot(h1, wh2_ref[...],
                         preferred_element_type=jnp.float32) + bh2_ref[...]
        logits_ref[...] = logits
        m = jnp.max(logits, axis=1, keepdims=True)
        e = jnp.exp(logits - m)
        probs_ref[...] = e / jnp.sum(e, axis=1, keepdims=True)


def _full(shape):
    return pl.BlockSpec(shape, lambda l, i: tuple(0 for _ in shape))


def _pad(w, rows, cols):
    return jnp.zeros((rows, cols), w.dtype).at[:w.shape[0], :w.shape[1]].set(w)


def kernel(node_feat, adj_matrix, W1, b1, W2, b2, W3, b3, Wh1, bh1, Wh2, bh2):
    adj16, d = pl.pallas_call(
        _cast_deg_kernel,
        grid=(N // _CAST_BM,),
        in_specs=[pl.BlockSpec((_CAST_BM, N), lambda i: (i, 0))],
        out_specs=[
            pl.BlockSpec((_CAST_BM, N), lambda i: (i, 0)),
            pl.BlockSpec((_CAST_BM, 1), lambda i: (i, 0)),
        ],
        out_shape=[
            jax.ShapeDtypeStruct((N, N), jnp.float8_e4m3fn),
            jax.ShapeDtypeStruct((N, 1), jnp.float32),
        ],
    )(adj_matrix)

    w_stack = jnp.stack([_pad(W1, _DIN, _DOUT), _pad(W2, _DIN, _DOUT),
                         _pad(W3, _DIN, _DOUT)])
    b_stack = jnp.stack([_pad(b1.reshape(1, -1), 1, _DOUT),
                         _pad(b2.reshape(1, -1), 1, _DOUT),
                         _pad(b3.reshape(1, -1), 1, _DOUT)])

    ni = N // _BM
    wh1 = _pad(Wh1, _DOUT, Wh1.shape[1])
    logits, probs = pl.pallas_call(
        functools.partial(_gcn_kernel, bm=_BM, ni=ni),
        grid=(3, ni),
        in_specs=[
            pl.BlockSpec((_BM, N), lambda l, i: (i, 0)),
            _full((N, _DIN)),
            pl.BlockSpec((1, _DIN, _DOUT), lambda l, i: (l, 0, 0)),
            pl.BlockSpec((1, 1, _DOUT), lambda l, i: (l, 0, 0)),
            _full((N, 1)),
            _full(wh1.shape),
            _full((1, Wh1.shape[1])),
            _full(Wh2.shape),
            _full((1, 2)),
        ],
        out_specs=[_full((1, 2)), _full((1, 2))],
        out_shape=[
            jax.ShapeDtypeStruct((1, 2), jnp.float32),
            jax.ShapeDtypeStruct((1, 2), jnp.float32),
        ],
        scratch_shapes=[
            pltpu.VMEM((N, _DIN), jnp.float32),
            pltpu.VMEM((N, _DOUT), jnp.float32),
            pltpu.VMEM((N, _DOUT), jnp.bfloat16),
            pltpu.VMEM((1, _DOUT), jnp.float32),
        ],
    )(adj16, node_feat, w_stack, b_stack, d,
      wh1, bh1.reshape(1, -1), Wh2, bh2.reshape(1, -1))

    return (logits.reshape(2), probs.reshape(2))


# speedup vs baseline: 1.4218x; 1.0030x over previous
import functools

import jax
import jax.numpy as jnp
from jax.experimental import pallas as pl
from jax.experimental.pallas import tpu as pltpu

N = 8192
_DIN = 128
_DOUT = 64
_CAST_BM = 256
_BM = 1024


def _cast_deg_kernel(adj_ref, adj8_ref, d_ref):
    blk = adj_ref[...]
    adj8_ref[...] = blk.astype(jnp.float8_e4m3fn)
    deg = jnp.sum(blk, axis=1, keepdims=True) + 1.0
    d_ref[...] = jax.lax.rsqrt(jnp.maximum(deg, 1e-12))


def _gcn_kernel(adj16_ref, nf_ref, w_ref, b_ref, d_ref,
                wh1_ref, bh1_ref, wh2_ref, bh2_ref,
                logits_ref, probs_ref,
                h_ref, z_ref, zb_ref, emb_ref, *, bm, ni):
    l = pl.program_id(0)
    i = pl.program_id(1)

    @pl.when((l == 0) & (i == 0))
    def _():
        h_ref[...] = nf_ref[...]
        emb_ref[...] = jnp.zeros_like(emb_ref)

    @pl.when(i == 0)
    def _():
        z = d_ref[...] * jnp.dot(h_ref[...], w_ref[0],
                                 preferred_element_type=jnp.float32)
        z_ref[...] = z
        zb_ref[...] = z.astype(jnp.bfloat16)

    row0 = i * bm
    t = jax.lax.dot_general(
        adj16_ref[...], zb_ref[...], (((1,), (0,)), ((), ())),
        preferred_element_type=jnp.float32)
    t = t + z_ref[pl.ds(row0, bm), :]
    di = d_ref[pl.ds(row0, bm), :]
    y = jnp.maximum(di * t + b_ref[0], 0.0)
    h_ref[pl.ds(row0, bm), :_DOUT] = y

    @pl.when(l == 2)
    def _():
        emb_ref[...] += jnp.sum(y, axis=0, keepdims=True)

    @pl.when((l == 2) & (i == ni - 1))
    def _():
        emb = emb_ref[...] * (1.0 / N)
        h1 = jnp.dot(emb, wh1_ref[...],
                     preferred_element_type=jnp.float32) + bh1_ref[...]
        u = jnp.exp(h1)
        lg = jnp.log(jnp.where(u == 1.0, 2.0, u))
        em1 = jnp.where(u == 1.0, h1, (u - 1.0) * h1 / lg)
        h1 = jnp.where(h1 > 0, h1, em1)
        logits = jnp.dot(h1, wh2_ref[...],
                         preferred_element_type=jnp.float32) + bh2_ref[...]
        logits_ref[...] = logits
        m = jnp.max(logits, axis=1, keepdims=True)
        e = jnp.exp(logits - m)
        probs_ref[...] = e / jnp.sum(e, axis=1, keepdims=True)


def _full(shape):
    return pl.BlockSpec(shape, lambda l, i: tuple(0 for _ in shape))


def _pad(w, rows, cols):
    return jnp.zeros((rows, cols), w.dtype).at[:w.shape[0], :w.shape[1]].set(w)


def kernel(node_feat, adj_matrix, W1, b1, W2, b2, W3, b3, Wh1, bh1, Wh2, bh2):
    adj16, d = pl.pallas_call(
        _cast_deg_kernel,
        grid=(N // _CAST_BM,),
        in_specs=[pl.BlockSpec((_CAST_BM, N), lambda i: (i, 0))],
        out_specs=[
            pl.BlockSpec((_CAST_BM, N), lambda i: (i, 0)),
            pl.BlockSpec((_CAST_BM, 1), lambda i: (i, 0)),
        ],
        out_shape=[
            jax.ShapeDtypeStruct((N, N), jnp.float8_e4m3fn),
            jax.ShapeDtypeStruct((N, 1), jnp.float32),
        ],
        compiler_params=pltpu.CompilerParams(
            dimension_semantics=("parallel",)),
    )(adj_matrix)

    w_stack = jnp.stack([_pad(W1, _DIN, _DOUT), _pad(W2, _DIN, _DOUT),
                         _pad(W3, _DIN, _DOUT)])
    b_stack = jnp.stack([_pad(b1.reshape(1, -1), 1, _DOUT),
                         _pad(b2.reshape(1, -1), 1, _DOUT),
                         _pad(b3.reshape(1, -1), 1, _DOUT)])

    ni = N // _BM
    wh1 = _pad(Wh1, _DOUT, Wh1.shape[1])
    logits, probs = pl.pallas_call(
        functools.partial(_gcn_kernel, bm=_BM, ni=ni),
        grid=(3, ni),
        in_specs=[
            pl.BlockSpec((_BM, N), lambda l, i: (i, 0)),
            _full((N, _DIN)),
            pl.BlockSpec((1, _DIN, _DOUT), lambda l, i: (l, 0, 0)),
            pl.BlockSpec((1, 1, _DOUT), lambda l, i: (l, 0, 0)),
            _full((N, 1)),
            _full(wh1.shape),
            _full((1, Wh1.shape[1])),
            _full(Wh2.shape),
            _full((1, 2)),
        ],
        out_specs=[_full((1, 2)), _full((1, 2))],
        out_shape=[
            jax.ShapeDtypeStruct((1, 2), jnp.float32),
            jax.ShapeDtypeStruct((1, 2), jnp.float32),
        ],
        scratch_shapes=[
            pltpu.VMEM((N, _DIN), jnp.float32),
            pltpu.VMEM((N, _DOUT), jnp.float32),
            pltpu.VMEM((N, _DOUT), jnp.bfloat16),
            pltpu.VMEM((1, _DOUT), jnp.float32),
        ],
    )(adj16, node_feat, w_stack, b_stack, d,
      wh1, bh1.reshape(1, -1), Wh2, bh2.reshape(1, -1))

    return (logits.reshape(2), probs.reshape(2))
